# baseline (speedup 1.0000x reference)
import functools

import jax
import jax.numpy as jnp
from jax.experimental import pallas as pl
from jax.experimental.pallas import tpu as pltpu
from jax.experimental.pallas import tpu_sc as plsc

_NC = 2
_NS = 16
_NW = _NC * _NS
_C = 160


def _gather_rows(idx_grouped, table, n):
    d = table.shape[1]
    nw, per_w = idx_grouped.shape
    c = _C
    g_chunks = per_w // c
    mesh = plsc.VectorSubcoreMesh(core_axis_name="c", subcore_axis_name="s")

    @functools.partial(
        pl.kernel,
        out_type=jax.ShapeDtypeStruct((n, d), table.dtype),
        mesh=mesh,
        scratch_types=[
            pltpu.VMEM((per_w,), jnp.int32),
            pltpu.VMEM((2, c, d), table.dtype),
            pltpu.VMEM_SHARED(table.shape, table.dtype),
            pltpu.VMEM_SHARED((_NS, 2, c, d), table.dtype),
            pltpu.SemaphoreType.DMA,
            pltpu.SemaphoreType.DMA,
            pltpu.SemaphoreType.DMA,
            pltpu.SemaphoreType.DMA,
            pltpu.SemaphoreType.DMA,
            pltpu.SemaphoreType.DMA,
        ],
    )
    def k(table_hbm, idx_hbm, out_hbm, idx_v, rows_v, table_sp, stage_sp,
          gsem0, gsem1, ssem0, ssem1, wsem0, wsem1):
        sid = jax.lax.axis_index("s")
        wid = sid * _NC + jax.lax.axis_index("c")
        base = wid * per_w
        t_rows = table.shape[0] // _NS
        pltpu.sync_copy(
            table_hbm.at[pl.ds(sid * t_rows, t_rows)],
            table_sp.at[pl.ds(sid * t_rows, t_rows)],
        )
        pltpu.sync_copy(idx_hbm.at[wid], idx_v)
        plsc.subcore_barrier()

        gsems = (gsem0, gsem1)
        ssems = (ssem0, ssem1)
        wsems = (wsem0, wsem1)

        def start_gather(chunk, buf):
            pltpu.async_copy(
                table_sp.at[idx_v.at[pl.ds(chunk * c, c)]],
                rows_v.at[buf],
                gsems[buf],
            )

        def process_chunk(chunk, buf):
            pltpu.make_async_copy(
                table_sp.at[idx_v.at[pl.ds(0, c)]], rows_v.at[buf], gsems[buf]
            ).wait()
            pltpu.async_copy(rows_v.at[buf], stage_sp.at[sid, buf], ssems[buf])
            pltpu.make_async_copy(
                rows_v.at[buf], stage_sp.at[sid, buf], ssems[buf]
            ).wait()
            pltpu.async_copy(
                stage_sp.at[sid, buf],
                out_hbm.at[pl.ds(base + chunk * c, c)],
                wsems[buf],
            )

        def wait_write(buf):
            pltpu.make_async_copy(
                stage_sp.at[sid, buf],
                out_hbm.at[pl.ds(base, c)],
                wsems[buf],
            ).wait()

        start_gather(0, 0)
        start_gather(1, 1)
        process_chunk(0, 0)

        @pl.loop(2, g_chunks, step=2)
        def _(chunk0):
            for buf in (0, 1):
                chunk = chunk0 + buf
                other = 1 - buf
                process_chunk(chunk - 1, other)
                wait_write(buf)
                start_gather(chunk, buf)

        last = g_chunks - 1
        process_chunk(last, last % 2)
        wait_write(0)
        wait_write(1)

    return k(table, idx_grouped)


def kernel(input_pos_tensors, table):
    b, s = input_pos_tensors.shape
    n = b * s
    d = table.shape[1]
    idx_grouped = input_pos_tensors.T.reshape(_NW, n // _NW)
    out = _gather_rows(idx_grouped, table, n)
    return jnp.swapaxes(out.reshape(s, b, d), 0, 1)


# speedup vs baseline: 2.0926x; 2.0926x over previous
import functools

import jax
import jax.numpy as jnp
from jax.experimental import pallas as pl
from jax.experimental.pallas import tpu as pltpu
from jax.experimental.pallas import tpu_sc as plsc

_NC = 2
_NS = 16
_NW = _NC * _NS
_C = 200


def _gather_rows(idx_grouped, table, n):
    d = table.shape[1]
    nw, per_w = idx_grouped.shape
    c = _C
    g_chunks = per_w // c
    mesh = plsc.VectorSubcoreMesh(core_axis_name="c", subcore_axis_name="s")

    @functools.partial(
        pl.kernel,
        out_type=jax.ShapeDtypeStruct((n, d), table.dtype),
        mesh=mesh,
        scratch_types=[
            pltpu.VMEM((per_w,), jnp.int32),
            pltpu.VMEM((4, c, d), table.dtype),
            pltpu.VMEM_SHARED(table.shape, table.dtype),
            pltpu.SemaphoreType.DMA,
            pltpu.SemaphoreType.DMA,
            pltpu.SemaphoreType.DMA,
            pltpu.SemaphoreType.DMA,
            pltpu.SemaphoreType.DMA,
            pltpu.SemaphoreType.DMA,
            pltpu.SemaphoreType.DMA,
            pltpu.SemaphoreType.DMA,
        ],
    )
    def k(table_hbm, idx_hbm, out_hbm, idx_v, rows_v, table_sp,
          gsem0, gsem1, gsem2, gsem3, wsem0, wsem1, wsem2, wsem3):
        sid = jax.lax.axis_index("s")
        wid = sid * _NC + jax.lax.axis_index("c")
        base = wid * per_w
        t_rows = table.shape[0] // _NS
        pltpu.sync_copy(
            table_hbm.at[pl.ds(sid * t_rows, t_rows)],
            table_sp.at[pl.ds(sid * t_rows, t_rows)],
        )
        pltpu.sync_copy(idx_hbm.at[wid], idx_v)
        plsc.subcore_barrier()

        gsems = (gsem0, gsem1, gsem2, gsem3)
        wsems = (wsem0, wsem1, wsem2, wsem3)

        def start_gather(chunk, buf):
            pltpu.async_copy(
                table_sp.at[idx_v.at[pl.ds(chunk * c, c)]],
                rows_v.at[buf],
                gsems[buf],
            )

        def finish_chunk(chunk, buf):
            pltpu.make_async_copy(
                table_sp.at[idx_v.at[pl.ds(0, c)]], rows_v.at[buf], gsems[buf]
            ).wait()
            pltpu.async_copy(
                rows_v.at[buf],
                out_hbm.at[pl.ds(base + chunk * c, c)],
                wsems[buf],
            )

        def wait_write(buf):
            pltpu.make_async_copy(
                rows_v.at[buf],
                out_hbm.at[pl.ds(base, c)],
                wsems[buf],
            ).wait()

        for i in range(4):
            start_gather(i, i)
        for i in range(3):
            finish_chunk(i, i)

        @pl.loop(4, g_chunks, step=4)
        def _(chunk0):
            for buf in (0, 1, 2, 3):
                chunk = chunk0 + buf
                other = (buf + 3) % 4
                finish_chunk(chunk - 1, other)
                wait_write(buf)
                start_gather(chunk, buf)

        last = g_chunks - 1
        finish_chunk(last, last % 4)
        for i in range(4):
            wait_write(i)

    return k(table, idx_grouped)


def kernel(input_pos_tensors, table):
    b, s = input_pos_tensors.shape
    n = b * s
    d = table.shape[1]
    idx_grouped = input_pos_tensors.T.reshape(_NW, n // _NW)
    out = _gather_rows(idx_grouped, table, n)
    return jnp.swapaxes(out.reshape(s, b, d), 0, 1)
